# Initial kernel scaffold; baseline (speedup 1.0000x reference)
import functools

import jax
import jax.numpy as jnp
from jax import lax
from jax.experimental import pallas as pl
from jax.experimental.pallas import tpu as pltpu
from jax.experimental.pallas import tpu_sc as plsc

N = 100000
D = 128
B = 128
MARGIN = 0.0
EPS = 1e-06

NW = 32
RPW = N // NW
CH = 125
NCH = RPW // CH
IDXPAD = 3128
ACC = 3 * B * D


def _sc_pool_body(ab_hbm, pb_hbm, nb_hbm, a_hbm, p_hbm, n_hbm,
                  out_hbm, acc, buf, idxb, sem0, sem1):
    wid = lax.axis_index("c") * 16 + lax.axis_index("s")
    sems = (sem0, sem1)

    def _zero(i, _):
        acc[pl.ds(i * 16, 16)] = jnp.zeros((16,), jnp.float32)
        return 0
    lax.fori_loop(0, ACC // 16, _zero, 0)

    pltpu.sync_copy(ab_hbm.at[wid], idxb.at[0])
    pltpu.sync_copy(pb_hbm.at[wid], idxb.at[1])
    pltpu.sync_copy(nb_hbm.at[wid], idxb.at[2])

    xs = (a_hbm, p_hbm, n_hbm)
    steps = [(t, g) for t in range(3) for g in range(NCH)]

    def _start(c, pb):
        t, g = steps[c]
        row0 = wid * RPW + g * CH
        return pltpu.async_copy(
            xs[t].at[pl.ds(row0, CH), :], buf.at[pb], sems[pb])

    copies = [None] * len(steps)
    copies[0] = _start(0, 0)
    for c, (t, g) in enumerate(steps):
        pb = c % 2
        if c + 1 < len(steps):
            copies[c + 1] = _start(c + 1, (c + 1) % 2)
        copies[c].wait()
        t_off = t * B * D

        def _row(i, _, t=t, g=g, pb=pb, t_off=t_off):
            b = idxb[t, g * CH + i]
            base = t_off + b * D
            for j in range(8):
                v = buf[pb, i, pl.ds(16 * j, 16)]
                plsc.addupdate(acc.at[pl.ds(base + 16 * j, 16)], v)
            return 0
        lax.fori_loop(0, CH, _row, 0)

    pltpu.sync_copy(acc, out_hbm.at[wid])


_sc_pool = functools.partial(
    pl.kernel,
    out_type=jax.ShapeDtypeStruct((NW, ACC), jnp.float32),
    mesh=plsc.VectorSubcoreMesh(core_axis_name="c", subcore_axis_name="s"),
    scratch_types=[
        pltpu.VMEM((ACC,), jnp.float32),
        pltpu.VMEM((2, CH, D), jnp.float32),
        pltpu.VMEM((3, IDXPAD), jnp.int32),
        pltpu.SemaphoreType.DMA,
        pltpu.SemaphoreType.DMA,
    ],
)(_sc_pool_body)


def _loss_body(part_ref, agt_ref, pgt_ref, ngt_ref, out_ref):
    x = part_ref[...]
    pooled = jnp.sum(x, axis=0)
    a_p = pooled[0:B, :]
    p_p = pooled[B:2 * B, :]
    n_p = pooled[2 * B:3 * B, :]
    pos_d = jnp.sqrt(jnp.sum((p_p - a_p) ** 2, axis=1, keepdims=True))
    neg_d = jnp.sqrt(jnp.sum((n_p - a_p) ** 2, axis=1, keepdims=True))
    agt = agt_ref[...]
    coeff = jnp.abs(ngt_ref[...] - agt) / (jnp.abs(pgt_ref[...] - agt) + EPS)
    loss = jnp.maximum(pos_d - coeff * neg_d + MARGIN, 0.0)
    out_ref[0, 0] = jnp.sum(loss) / B


_loss = pl.pallas_call(
    _loss_body,
    out_shape=jax.ShapeDtypeStruct((1, 1), jnp.float32),
)


def _prep_idx(b):
    b = b.astype(jnp.int32).reshape(NW, RPW)
    return jnp.pad(b, ((0, 0), (0, IDXPAD - RPW)))


def kernel(anchor_batch, negative_batch, positive_batch, anchor, negative,
           positive, anchor_gt, negative_gt, positive_gt):
    ab = _prep_idx(anchor_batch)
    pb = _prep_idx(positive_batch)
    nb = _prep_idx(negative_batch)
    parts = _sc_pool(ab, pb, nb, anchor, positive, negative)
    parts = parts.reshape(NW, 3 * B, D)
    out = _loss(parts,
                anchor_gt.reshape(B, 1),
                positive_gt.reshape(B, 1),
                negative_gt.reshape(B, 1))
    return out[0, 0]


# speedup vs baseline: 4.1663x; 4.1663x over previous
import functools

import jax
import jax.numpy as jnp
import numpy as np
from jax import lax
from jax.experimental import pallas as pl
from jax.experimental.pallas import tpu as pltpu
from jax.experimental.pallas import tpu_sc as plsc

N = 100000
D = 128
B = 128
MARGIN = 0.0
EPS = 1e-06

NC = 2
NS = 16
NW = NC * NS
CHM = 128
NKM = 24
CHT = 56
SPAN = NKM * CHM + CHT
DUMMY = 3 * B

_START = [3120 * w + 8 * max(0, w - 12) for w in range(NW)]
_VALID = [3120 if w < 12 else 3128 for w in range(NW)]


def _sc_pool_body(im_hbm, it_hbm, a_hbm, p_hbm, n_hbm, out_hbm,
                  acc_sh, buf, idxm, idxt, zbuf, sem0, sem1):
    cid = lax.axis_index("c")
    sid = lax.axis_index("s")
    wid = cid * NS + sid
    s0 = 3120 * wid + 8 * jnp.maximum(0, wid - 12)
    sems = (sem0, sem1)

    def _z(i, _):
        zbuf[i // 8, pl.ds((i % 8) * 16, 16)] = jnp.zeros((16,), jnp.float32)
        return 0
    lax.fori_loop(0, B * 8, _z, 0)

    @pl.when(sid == 0)
    def _():
        for t in range(3):
            pltpu.sync_copy(zbuf, acc_sh.at[pl.ds(t * B, B), :])
        pltpu.sync_copy(zbuf.at[pl.ds(0, 8), :],
                        acc_sh.at[pl.ds(3 * B, 8), :])

    plsc.subcore_barrier()

    pltpu.sync_copy(im_hbm.at[wid], idxm)
    pltpu.sync_copy(it_hbm.at[wid], idxt)

    xs = (a_hbm, p_hbm, n_hbm)
    steps = [(t, k) for t in range(3) for k in range(NKM + 1)]

    def _start(c, pb):
        t, k = steps[c]
        sz = CHM if k < NKM else CHT
        row0 = s0 + CHM * k
        return pltpu.async_copy(xs[t].at[pl.ds(row0, sz), :],
                                buf.at[pb, pl.ds(0, sz), :], sems[pb])

    copies = [None] * len(steps)
    copies[0] = _start(0, 0)
    for c, (t, k) in enumerate(steps):
        pb = c % 2
        if c + 1 < len(steps):
            copies[c + 1] = _start(c + 1, (c + 1) % 2)
        copies[c].wait()
        if k < NKM:
            pltpu.sync_copy(buf.at[pb], acc_sh.at[idxm.at[t, k]], add=True)
        else:
            pltpu.sync_copy(buf.at[pb, pl.ds(0, CHT), :],
                            acc_sh.at[idxt.at[t]], add=True)

    plsc.subcore_barrier()

    @pl.when(sid == 0)
    def _():
        pltpu.sync_copy(acc_sh.at[pl.ds(0, 3 * B), :], out_hbm.at[cid])


_sc_pool = functools.partial(
    pl.kernel,
    out_type=jax.ShapeDtypeStruct((NC, 3 * B, D), jnp.float32),
    mesh=plsc.VectorSubcoreMesh(core_axis_name="c", subcore_axis_name="s"),
    scratch_types=[
        pltpu.VMEM_SHARED((3 * B + 8, D), jnp.float32),
        pltpu.VMEM((2, CHM, D), jnp.float32),
        pltpu.VMEM((3, NKM, CHM), jnp.int32),
        pltpu.VMEM((8, CHT), jnp.int32),
        pltpu.VMEM((B, D), jnp.float32),
        pltpu.SemaphoreType.DMA,
        pltpu.SemaphoreType.DMA,
    ],
)(_sc_pool_body)


def _loss_body(part_ref, agt_ref, pgt_ref, ngt_ref, out_ref):
    pooled = part_ref[0] + part_ref[1]
    a_p = pooled[0:B, :]
    p_p = pooled[B:2 * B, :]
    n_p = pooled[2 * B:3 * B, :]
    pos_d = jnp.sqrt(jnp.sum((p_p - a_p) ** 2, axis=1, keepdims=True))
    neg_d = jnp.sqrt(jnp.sum((n_p - a_p) ** 2, axis=1, keepdims=True))
    agt = agt_ref[...]
    coeff = jnp.abs(ngt_ref[...] - agt) / (jnp.abs(pgt_ref[...] - agt) + EPS)
    loss = jnp.maximum(pos_d - coeff * neg_d + MARGIN, 0.0)
    out_ref[...] = (jnp.sum(loss) / B).reshape(1, 1)


_loss = pl.pallas_call(
    _loss_body,
    out_shape=jax.ShapeDtypeStruct((1, 1), jnp.float32),
)


def _prep_idx(ab, pb, nb):
    pos = jnp.asarray(np.array(_START)[:, None] + np.arange(SPAN)[None, :])
    valid = jnp.asarray(np.array(_VALID))[:, None]
    tail_j = jnp.arange(NKM * CHM, SPAN)[None, :]
    mains, tails = [], []
    for t, b in enumerate((ab, pb, nb)):
        arr = b.astype(jnp.int32) + t * B
        g = arr[pos]
        mains.append(g[:, :NKM * CHM].reshape(NW, NKM, CHM))
        tails.append(jnp.where(tail_j < valid, g[:, NKM * CHM:], DUMMY))
    idx_main = jnp.stack(mains, axis=1)
    tail = jnp.stack(tails, axis=1)
    pad = jnp.full((NW, 5, CHT), DUMMY, jnp.int32)
    idx_tail = jnp.concatenate([tail, pad], axis=1)
    return idx_main, idx_tail


def kernel(anchor_batch, negative_batch, positive_batch, anchor, negative,
           positive, anchor_gt, negative_gt, positive_gt):
    idx_main, idx_tail = _prep_idx(anchor_batch, positive_batch,
                                   negative_batch)
    parts = _sc_pool(idx_main, idx_tail, anchor, positive, negative)
    out = _loss(parts,
                anchor_gt.reshape(B, 1),
                positive_gt.reshape(B, 1),
                negative_gt.reshape(B, 1))
    return out[0, 0]


# speedup vs baseline: 5.3814x; 1.2917x over previous
import functools

import jax
import jax.numpy as jnp
from jax import lax
from jax.experimental import pallas as pl
from jax.experimental.pallas import tpu as pltpu
from jax.experimental.pallas import tpu_sc as plsc

N = 100000
D = 128
B = 128
MARGIN = 0.0
EPS = 1e-06

NC = 2
NS = 16
NW = NC * NS
CHM = 128
NKM = 24
CHT = 56
SPAN = NKM * CHM + CHT
DUMMY = 3 * B


def _sc_pool_body(im_hbm, it_hbm, a_hbm, p_hbm, n_hbm, out_hbm,
                  acc_sh, buf, idxm, idxt, zbuf, sem0, sem1):
    cid = lax.axis_index("c")
    sid = lax.axis_index("s")
    wid = cid * NS + sid
    s0 = 3120 * wid + 8 * jnp.maximum(0, wid - 12)
    sems = (sem0, sem1)

    def _z(i, _):
        zbuf[i // 8, pl.ds((i % 8) * 16, 16)] = jnp.zeros((16,), jnp.float32)
        return 0
    lax.fori_loop(0, B * 8, _z, 0)

    @pl.when(sid == 0)
    def _():
        for t in range(3):
            pltpu.sync_copy(zbuf, acc_sh.at[pl.ds(t * B, B), :])
        pltpu.sync_copy(zbuf.at[pl.ds(0, 8), :],
                        acc_sh.at[pl.ds(3 * B, 8), :])

    plsc.subcore_barrier()

    pltpu.sync_copy(im_hbm.at[wid], idxm)
    pltpu.sync_copy(it_hbm.at[wid], idxt)

    xs = (a_hbm, p_hbm, n_hbm)
    steps = [(t, k) for t in range(3) for k in range(NKM + 1)]

    def _start(c, pb):
        t, k = steps[c]
        sz = CHM if k < NKM else CHT
        row0 = s0 + CHM * k
        return pltpu.async_copy(xs[t].at[pl.ds(row0, sz), :],
                                buf.at[pb, pl.ds(0, sz), :], sems[pb])

    copies = [None] * len(steps)
    copies[0] = _start(0, 0)
    for c, (t, k) in enumerate(steps):
        pb = c % 2
        if c + 1 < len(steps):
            copies[c + 1] = _start(c + 1, (c + 1) % 2)
        copies[c].wait()
        if k < NKM:
            pltpu.sync_copy(buf.at[pb], acc_sh.at[idxm.at[t, k]], add=True)
        else:
            pltpu.sync_copy(buf.at[pb, pl.ds(0, CHT), :],
                            acc_sh.at[idxt.at[t]], add=True)

    plsc.subcore_barrier()

    @pl.when(sid == 0)
    def _():
        pltpu.sync_copy(acc_sh.at[pl.ds(0, 3 * B), :], out_hbm.at[cid])


_sc_pool = functools.partial(
    pl.kernel,
    out_type=jax.ShapeDtypeStruct((NC, 3 * B, D), jnp.float32),
    mesh=plsc.VectorSubcoreMesh(core_axis_name="c", subcore_axis_name="s"),
    scratch_types=[
        pltpu.VMEM_SHARED((3 * B + 8, D), jnp.float32),
        pltpu.VMEM((2, CHM, D), jnp.float32),
        pltpu.VMEM((3, NKM, CHM), jnp.int32),
        pltpu.VMEM((8, CHT), jnp.int32),
        pltpu.VMEM((B, D), jnp.float32),
        pltpu.SemaphoreType.DMA,
        pltpu.SemaphoreType.DMA,
    ],
)(_sc_pool_body)


def _loss_body(part_ref, agt_ref, pgt_ref, ngt_ref, out_ref):
    pooled = part_ref[0] + part_ref[1]
    a_p = pooled[0:B, :]
    p_p = pooled[B:2 * B, :]
    n_p = pooled[2 * B:3 * B, :]
    pos_d = jnp.sqrt(jnp.sum((p_p - a_p) ** 2, axis=1, keepdims=True))
    neg_d = jnp.sqrt(jnp.sum((n_p - a_p) ** 2, axis=1, keepdims=True))
    agt = agt_ref[...]
    coeff = jnp.abs(ngt_ref[...] - agt) / (jnp.abs(pgt_ref[...] - agt) + EPS)
    loss = jnp.maximum(pos_d - coeff * neg_d + MARGIN, 0.0)
    out_ref[...] = (jnp.sum(loss) / B).reshape(1, 1)


_loss = pl.pallas_call(
    _loss_body,
    out_shape=jax.ShapeDtypeStruct((1, 1), jnp.float32),
)


def _prep_idx(ab, pb, nb):
    mains, tails = [], []
    dummy8 = jnp.full((12, 8), DUMMY, jnp.int32)
    for t, b in enumerate((ab, pb, nb)):
        arr = b.astype(jnp.int32) + t * B
        lo = arr[:12 * 3120].reshape(12, 3120)
        hi = arr[12 * 3120:].reshape(20, 3128)
        main = jnp.concatenate([lo[:, :NKM * CHM], hi[:, :NKM * CHM]])
        mains.append(main.reshape(NW, NKM, CHM))
        tail_lo = jnp.concatenate([lo[:, NKM * CHM:], dummy8], axis=1)
        tails.append(jnp.concatenate([tail_lo, hi[:, NKM * CHM:]]))
    idx_main = jnp.stack(mains, axis=1)
    tail = jnp.stack(tails, axis=1)
    pad = jnp.full((NW, 5, CHT), DUMMY, jnp.int32)
    idx_tail = jnp.concatenate([tail, pad], axis=1)
    return idx_main, idx_tail


def kernel(anchor_batch, negative_batch, positive_batch, anchor, negative,
           positive, anchor_gt, negative_gt, positive_gt):
    idx_main, idx_tail = _prep_idx(anchor_batch, positive_batch,
                                   negative_batch)
    parts = _sc_pool(idx_main, idx_tail, anchor, positive, negative)
    out = _loss(parts,
                anchor_gt.reshape(B, 1),
                positive_gt.reshape(B, 1),
                negative_gt.reshape(B, 1))
    return out[0, 0]
